# Initial kernel scaffold; baseline (speedup 1.0000x reference)
import jax
import jax.numpy as jnp
from jax.experimental import pallas as pl


def kernel(x, gate_weight, gate_bias):
    raise NotImplementedError("write your pallas kernel here")


# speedup vs baseline: 1.1086x; 1.1086x over previous
import functools

import jax
import jax.numpy as jnp
from jax.experimental import pallas as pl

_INPUT_DIM = 4096
_NUM_TOWERS = 64
_TOP_K = 8
_BLOCK_ROWS = 1024


def _router_body(x_ref, w_ref, b_ref, scores_ref, idx_ref):
    logits = jnp.dot(x_ref[...], w_ref[...], preferred_element_type=jnp.float32)
    logits = logits + b_ref[...]
    col = jax.lax.broadcasted_iota(jnp.int32, logits.shape, 1)
    vals = logits
    tops = []
    args = []
    for _ in range(_TOP_K):
        m = jnp.max(vals, axis=1, keepdims=True)
        cand = jnp.where(vals == m, col, _NUM_TOWERS)
        a = jnp.min(cand, axis=1, keepdims=True)
        tops.append(m)
        args.append(a)
        vals = jnp.where(col == a, -jnp.inf, vals)
    top = jnp.concatenate(tops, axis=1)
    idx = jnp.concatenate(args, axis=1)
    e = jnp.exp(top - top[:, :1])
    scores_ref[...] = e / jnp.sum(e, axis=1, keepdims=True)
    idx_ref[...] = idx


@functools.partial(jax.jit, static_argnames=("interpret",))
def kernel(x, gate_weight, gate_bias, interpret=False):
    n_tokens = x.shape[0]
    wt = gate_weight.T
    b = gate_bias.reshape(1, _NUM_TOWERS)
    grid = (n_tokens // _BLOCK_ROWS,)
    scores, idx = pl.pallas_call(
        _router_body,
        grid=grid,
        in_specs=[
            pl.BlockSpec((_BLOCK_ROWS, _INPUT_DIM), lambda i: (i, 0)),
            pl.BlockSpec((_INPUT_DIM, _NUM_TOWERS), lambda i: (0, 0)),
            pl.BlockSpec((1, _NUM_TOWERS), lambda i: (0, 0)),
        ],
        out_specs=[
            pl.BlockSpec((_BLOCK_ROWS, _TOP_K), lambda i: (i, 0)),
            pl.BlockSpec((_BLOCK_ROWS, _TOP_K), lambda i: (i, 0)),
        ],
        out_shape=[
            jax.ShapeDtypeStruct((n_tokens, _TOP_K), jnp.float32),
            jax.ShapeDtypeStruct((n_tokens, _TOP_K), jnp.int32),
        ],
        interpret=interpret,
    )(x, wt, b)
    return scores, idx


# speedup vs baseline: 1.2682x; 1.1440x over previous
import functools

import jax
import jax.numpy as jnp
from jax.experimental import pallas as pl

_INPUT_DIM = 4096
_NUM_TOWERS = 64
_TOP_K = 8
_BLOCK_ROWS = 1024


def _router_body(x_ref, w_ref, b_ref, scores_ref, idx_ref):
    _INT_MIN = jnp.int32(-2147483648)
    logits = jnp.dot(x_ref[...], w_ref[...], preferred_element_type=jnp.float32)
    logits = logits + b_ref[...]
    s = jax.lax.bitcast_convert_type(logits, jnp.int32)
    key = jnp.where(s >= 0, s, _INT_MIN - s)
    col = jax.lax.broadcasted_iota(jnp.int32, logits.shape, 1)
    packed = (key & jnp.int32(~63)) | (jnp.int32(_NUM_TOWERS - 1) - col)
    ms = []
    for _ in range(_TOP_K):
        m = jnp.max(packed, axis=1, keepdims=True)
        ms.append(m)
        packed = jnp.where(packed == m, _INT_MIN, packed)
    mk = jnp.concatenate(ms, axis=1)
    idx = jnp.int32(_NUM_TOWERS - 1) - (mk & jnp.int32(63))
    kv = mk & jnp.int32(~63)
    sv = jnp.where(kv >= 0, kv, _INT_MIN - kv)
    top = jax.lax.bitcast_convert_type(sv, jnp.float32)
    e = jnp.exp(top - top[:, :1])
    scores_ref[...] = e / jnp.sum(e, axis=1, keepdims=True)
    idx_ref[...] = idx


@functools.partial(jax.jit, static_argnames=("interpret",))
def kernel(x, gate_weight, gate_bias, interpret=False):
    n_tokens = x.shape[0]
    wt = gate_weight.T
    b = gate_bias.reshape(1, _NUM_TOWERS)
    grid = (n_tokens // _BLOCK_ROWS,)
    scores, idx = pl.pallas_call(
        _router_body,
        grid=grid,
        in_specs=[
            pl.BlockSpec((_BLOCK_ROWS, _INPUT_DIM), lambda i: (i, 0)),
            pl.BlockSpec((_INPUT_DIM, _NUM_TOWERS), lambda i: (0, 0)),
            pl.BlockSpec((1, _NUM_TOWERS), lambda i: (0, 0)),
        ],
        out_specs=[
            pl.BlockSpec((_BLOCK_ROWS, _TOP_K), lambda i: (i, 0)),
            pl.BlockSpec((_BLOCK_ROWS, _TOP_K), lambda i: (i, 0)),
        ],
        out_shape=[
            jax.ShapeDtypeStruct((n_tokens, _TOP_K), jnp.float32),
            jax.ShapeDtypeStruct((n_tokens, _TOP_K), jnp.int32),
        ],
        interpret=interpret,
    )(x, wt, b)
    return scores, idx


# speedup vs baseline: 1.2793x; 1.0088x over previous
import functools

import jax
import jax.numpy as jnp
from jax.experimental import pallas as pl

_INPUT_DIM = 4096
_NUM_TOWERS = 64
_TOP_K = 8
_BLOCK_ROWS = 1024


def _router_body(x_ref, w_ref, b_ref, scores_ref, idx_ref):
    logits = jnp.dot(x_ref[...], w_ref[...], preferred_element_type=jnp.float32)
    logits = logits + b_ref[...]
    inv_col = (jnp.int32(_NUM_TOWERS - 1) - jax.lax.broadcasted_iota(
        jnp.int32, logits.shape, 1)).astype(jnp.float32)
    vals = logits
    tops = []
    pms = []
    for _ in range(_TOP_K):
        m = jnp.max(vals, axis=1, keepdims=True)
        t = jnp.where(vals == m, inv_col, jnp.float32(-1.0))
        pm = jnp.max(t, axis=1, keepdims=True)
        vals = jnp.where(inv_col == pm, -jnp.inf, vals)
        tops.append(m)
        pms.append(pm)
    top = jnp.concatenate(tops, axis=1)
    idx = jnp.int32(_NUM_TOWERS - 1) - jnp.concatenate(pms, axis=1).astype(jnp.int32)
    e = jnp.exp(top - top[:, :1])
    scores_ref[...] = e / jnp.sum(e, axis=1, keepdims=True)
    idx_ref[...] = idx


@functools.partial(jax.jit, static_argnames=("interpret",))
def kernel(x, gate_weight, gate_bias, interpret=False):
    n_tokens = x.shape[0]
    wt = gate_weight.T
    b = gate_bias.reshape(1, _NUM_TOWERS)
    grid = (n_tokens // _BLOCK_ROWS,)
    scores, idx = pl.pallas_call(
        _router_body,
        grid=grid,
        in_specs=[
            pl.BlockSpec((_BLOCK_ROWS, _INPUT_DIM), lambda i: (i, 0)),
            pl.BlockSpec((_INPUT_DIM, _NUM_TOWERS), lambda i: (0, 0)),
            pl.BlockSpec((1, _NUM_TOWERS), lambda i: (0, 0)),
        ],
        out_specs=[
            pl.BlockSpec((_BLOCK_ROWS, _TOP_K), lambda i: (i, 0)),
            pl.BlockSpec((_BLOCK_ROWS, _TOP_K), lambda i: (i, 0)),
        ],
        out_shape=[
            jax.ShapeDtypeStruct((n_tokens, _TOP_K), jnp.float32),
            jax.ShapeDtypeStruct((n_tokens, _TOP_K), jnp.int32),
        ],
        interpret=interpret,
    )(x, wt, b)
    return scores, idx


# speedup vs baseline: 1.4081x; 1.1007x over previous
import functools

import jax
import jax.numpy as jnp
from jax.experimental import pallas as pl

_INPUT_DIM = 4096
_NUM_TOWERS = 64
_TOP_K = 8
_BLOCK_ROWS = 1024


def _router_body(x_ref, w_ref, b_ref, scores_ref, idx_ref):
    logits = jnp.dot(x_ref[...], w_ref[...], preferred_element_type=jnp.float32)
    logits = logits + b_ref[...]
    s = jax.lax.bitcast_convert_type(logits, jnp.int32)
    col = jax.lax.broadcasted_iota(jnp.int32, logits.shape, 1)
    code = jnp.where(s >= 0, jnp.int32(_NUM_TOWERS - 1) - col, col)
    key = jax.lax.bitcast_convert_type((s & jnp.int32(~63)) | code,
                                       jnp.float32)
    ms = []
    for _ in range(_TOP_K):
        m = jnp.max(key, axis=1, keepdims=True)
        ms.append(m)
        key = jnp.where(key == m, -jnp.inf, key)
    mk = jax.lax.bitcast_convert_type(jnp.concatenate(ms, axis=1), jnp.int32)
    idx = jnp.where(mk >= 0, jnp.int32(_NUM_TOWERS - 1) - (mk & jnp.int32(63)),
                    mk & jnp.int32(63))
    top = jax.lax.bitcast_convert_type(mk & jnp.int32(~63), jnp.float32)
    e = jnp.exp(top - top[:, :1])
    scores_ref[...] = e / jnp.sum(e, axis=1, keepdims=True)
    idx_ref[...] = idx


@functools.partial(jax.jit, static_argnames=("interpret",))
def kernel(x, gate_weight, gate_bias, interpret=False):
    n_tokens = x.shape[0]
    wt = gate_weight.T
    b = gate_bias.reshape(1, _NUM_TOWERS)
    grid = (n_tokens // _BLOCK_ROWS,)
    scores, idx = pl.pallas_call(
        _router_body,
        grid=grid,
        in_specs=[
            pl.BlockSpec((_BLOCK_ROWS, _INPUT_DIM), lambda i: (i, 0)),
            pl.BlockSpec((_INPUT_DIM, _NUM_TOWERS), lambda i: (0, 0)),
            pl.BlockSpec((1, _NUM_TOWERS), lambda i: (0, 0)),
        ],
        out_specs=[
            pl.BlockSpec((_BLOCK_ROWS, _TOP_K), lambda i: (i, 0)),
            pl.BlockSpec((_BLOCK_ROWS, _TOP_K), lambda i: (i, 0)),
        ],
        out_shape=[
            jax.ShapeDtypeStruct((n_tokens, _TOP_K), jnp.float32),
            jax.ShapeDtypeStruct((n_tokens, _TOP_K), jnp.int32),
        ],
        interpret=interpret,
    )(x, wt, b)
    return scores, idx
